# baseline (speedup 1.0000x reference)
import functools

import jax
import jax.numpy as jnp
from jax import lax
from jax.experimental import pallas as pl
from jax.experimental.pallas import tpu as pltpu
from jax.experimental.pallas import tpu_sc as plsc

B, F, A, O = 16, 6, 8732, 16
BF = B * F
ASC = 8736
NCHUNK = ASC // 16
NP_RATIO = 3
THRESHOLD = 0.5
VAR0, VAR1 = 0.1, 0.2
NWORKERS = 32
FRAMES_PER_W = BF // NWORKERS


def _tc_body(truths_ref, anch_ref, loc_ref, conf_ref,
             lossl_ref, sumpos_ref, lc_ref, npos_ref, scr_ref):
    f = pl.program_id(0)

    @pl.when(f == 0)
    def _init():
        lossl_ref[...] = jnp.zeros_like(lossl_ref)
        sumpos_ref[...] = jnp.zeros_like(sumpos_ref)
        acx0 = anch_ref[0:1, :]
        acy0 = anch_ref[1:2, :]
        aw0 = anch_ref[2:3, :]
        ah0 = anch_ref[3:4, :]
        scr_ref[0:1, :] = acx0 - aw0 * 0.5
        scr_ref[1:2, :] = acy0 - ah0 * 0.5
        scr_ref[2:3, :] = acx0 + aw0 * 0.5
        scr_ref[3:4, :] = acy0 + ah0 * 0.5
        scr_ref[4:5, :] = aw0 * ah0
        scr_ref[5:6, :] = 1.0 / (VAR0 * aw0)
        scr_ref[6:7, :] = 1.0 / (VAR0 * ah0)
        scr_ref[7:8, :] = 1.0 / aw0
        scr_ref[8:9, :] = 1.0 / ah0

    th = truths_ref[...].reshape(O, 4)
    tx1 = th[:, 0:1]
    ty1 = th[:, 1:2]
    tx2 = th[:, 2:3]
    ty2 = th[:, 3:4]

    acx = anch_ref[0:1, :]
    acy = anch_ref[1:2, :]
    px1 = scr_ref[0:1, :]
    py1 = scr_ref[1:2, :]
    px2 = scr_ref[2:3, :]
    py2 = scr_ref[3:4, :]
    parea = scr_ref[4:5, :]
    inv_vw = scr_ref[5:6, :]
    inv_vh = scr_ref[6:7, :]
    inv_aw = scr_ref[7:8, :]
    inv_ah = scr_ref[8:9, :]

    iw = jnp.maximum(jnp.minimum(tx2, px2) - jnp.maximum(tx1, px1), 0.0)
    ih = jnp.maximum(jnp.minimum(ty2, py2) - jnp.maximum(ty1, py1), 0.0)
    inter = iw * ih
    tarea = (tx2 - tx1) * (ty2 - ty1)
    ov = inter / (tarea + parea - inter)

    bto = jnp.max(ov, axis=0, keepdims=True)
    bti = jnp.argmax(ov, axis=0).astype(jnp.int32)
    bti = bti.reshape(1, A)
    bpi = jnp.argmax(ov, axis=1).astype(jnp.int32)
    bpi = bpi.reshape(O, 1)

    lane16 = lax.broadcasted_iota(jnp.int32, (O, A), 1)
    trange = lax.broadcasted_iota(jnp.int32, (O, 1), 0)
    claimed = lane16 == bpi
    t_last = jnp.max(jnp.where(claimed, trange, -1), axis=0, keepdims=True)
    forced = t_last >= 0
    bto = jnp.where(forced, 2.0, bto)
    bti = jnp.where(forced, t_last, bti)

    pos = bto >= THRESHOLD
    posf = pos.astype(jnp.float32)

    oh = (bti == trange).astype(jnp.float32)
    mc = lax.dot_general(th, oh, (((0,), (0,)), ((), ())),
                         preferred_element_type=jnp.float32)
    mx1 = mc[0:1, :]
    my1 = mc[1:2, :]
    mx2 = mc[2:3, :]
    my2 = mc[3:4, :]

    gcx = ((mx1 + mx2) * 0.5 - acx) * inv_vw
    gcy = ((my1 + my2) * 0.5 - acy) * inv_vh
    gw = jnp.log((mx2 - mx1) * inv_aw) * (1.0 / VAR1)
    gh = jnp.log((my2 - my1) * inv_ah) * (1.0 / VAR1)

    def sl1(x):
        ax = jnp.abs(x)
        return jnp.where(ax < 1.0, 0.5 * x * x, ax - 0.5)

    lc4 = loc_ref[...].reshape(4, A)
    sl = (sl1(lc4[0:1, :] - gcx) + sl1(lc4[1:2, :] - gcy)
          + sl1(lc4[2:3, :] - gw) + sl1(lc4[3:4, :] - gh))
    lossl_ref[...] += jnp.sum(sl * posf, axis=1, keepdims=True)

    cf2 = conf_ref[...].reshape(2, A)
    c0 = cf2[0:1, :]
    c1 = cf2[1:2, :]
    mx = jnp.maximum(c0, c1)
    lse = mx + jnp.log(1.0 + jnp.exp(-jnp.abs(c0 - c1)))
    gathered = jnp.where(pos, c1, c0)
    ce = lse - gathered
    sumpos_ref[...] += jnp.sum(ce * posf, axis=1, keepdims=True)
    lc = jnp.where(pos, 0.0, ce)
    lc_ref[...] = jnp.concatenate(
        [lc, jnp.zeros((1, ASC - A), jnp.float32)], axis=1).reshape(1, 1, ASC)
    npos_ref[...] = jnp.broadcast_to(
        jnp.sum(posf, axis=1, keepdims=True), (1, 128)).reshape(1, 1, 128)


def _tc_stage(truths, anch_t, loc, conf):
    nf = truths.shape[0]
    return pl.pallas_call(
        _tc_body,
        grid=(nf,),
        in_specs=[
            pl.BlockSpec((1, O, 4), lambda f: (f, 0, 0)),
            pl.BlockSpec((4, A), lambda f: (0, 0)),
            pl.BlockSpec((1, 4, A), lambda f: (f, 0, 0)),
            pl.BlockSpec((1, 2, A), lambda f: (f, 0, 0)),
        ],
        out_specs=[
            pl.BlockSpec((1, 1), lambda f: (0, 0)),
            pl.BlockSpec((1, 1), lambda f: (0, 0)),
            pl.BlockSpec((1, 1, ASC), lambda f: (f, 0, 0)),
            pl.BlockSpec((1, 1, 128), lambda f: (f, 0, 0)),
        ],
        out_shape=[
            jax.ShapeDtypeStruct((1, 1), jnp.float32),
            jax.ShapeDtypeStruct((1, 1), jnp.float32),
            jax.ShapeDtypeStruct((nf, 1, ASC), jnp.float32),
            jax.ShapeDtypeStruct((nf, 1, 128), jnp.float32),
        ],
        scratch_shapes=[pltpu.VMEM((9, A), jnp.float32)],
    )(truths, anch_t, loc, conf)


def _sc_body(fpw, lc_hbm, np_hbm, out_hbm, vbuf, npbuf, outv):
    wid = lax.axis_index("s") * 2 + lax.axis_index("c")
    partial = jnp.zeros((16,), jnp.float32)
    lane0 = lax.broadcasted_iota(jnp.int32, (16,), 0) == 0
    for j in range(fpw):
        f = wid * fpw + j
        pltpu.sync_copy(lc_hbm.at[f], vbuf)
        pltpu.sync_copy(np_hbm.at[f], npbuf)
        npos = npbuf[pl.ds(0, 16)].astype(jnp.int32)
        k = jnp.minimum(npos * NP_RATIO, A - 1)

        def max_step(c, acc):
            return jnp.maximum(acc, vbuf[pl.ds(c * 16, 16)])

        vmax = lax.fori_loop(0, NCHUNK, max_step,
                             jnp.zeros((16,), jnp.float32), unroll=8)
        vmax = jnp.full((16,), jnp.max(vmax))

        def bs_step(_, carry):
            lo, hi = carry
            mid = (lo + hi) * 0.5

            def cnt_step(c, acc):
                m = vbuf[pl.ds(c * 16, 16)] >= mid
                return acc + plsc.all_reduce_population_count(m)

            cnt = lax.fori_loop(0, NCHUNK, cnt_step,
                                jnp.zeros((16,), jnp.int32), unroll=8)
            ok = cnt >= k
            lo = jnp.where(ok, mid, lo)
            hi = jnp.where(ok, hi, mid)
            return lo, hi

        lo0 = jnp.zeros((16,), jnp.float32)
        hi0 = vmax + 1.0
        t_f, _ = lax.fori_loop(0, 25, bs_step, (lo0, hi0))

        def fin_step(c, carry):
            cnt_gt, sum_gt = carry
            v = vbuf[pl.ds(c * 16, 16)]
            m = v > t_f
            cnt_gt = cnt_gt + plsc.all_reduce_population_count(m)
            sum_gt = sum_gt + jnp.where(m, v, 0.0)
            return cnt_gt, sum_gt

        cnt_gt, sum_gt = lax.fori_loop(
            0, NCHUNK, fin_step,
            (jnp.zeros((16,), jnp.int32), jnp.zeros((16,), jnp.float32)),
            unroll=8)
        rem = (k - cnt_gt).astype(jnp.float32) * t_f
        partial = partial + sum_gt + jnp.where(lane0, rem, 0.0)

    outv[...] = partial
    pltpu.sync_copy(outv, out_hbm.at[wid])


def _sc_stage(lc, npos):
    nf = lc.shape[0]
    fpw = nf // NWORKERS
    mesh = plsc.VectorSubcoreMesh(core_axis_name="c", subcore_axis_name="s")
    kfn = functools.partial(
        pl.kernel,
        mesh=mesh,
        out_type=jax.ShapeDtypeStruct((NWORKERS, 16), jnp.float32),
        scratch_types=[
            pltpu.VMEM((ASC,), jnp.float32),
            pltpu.VMEM((128,), jnp.float32),
            pltpu.VMEM((16,), jnp.float32),
        ],
        compiler_params=pltpu.CompilerParams(needs_layout_passes=False),
    )(functools.partial(_sc_body, fpw))
    return kfn(lc, npos)


SPLITS = (64, 32)


def kernel(loc_data, conf_data, anchors, targets):
    anch_t = anchors.T
    loc = loc_data.reshape(BF, A, 4)
    conf = conf_data.reshape(BF, A, 2)
    truths_all = targets[..., :4].reshape(BF, O, 4)

    loss_l = jnp.float32(0.0)
    loss_c = jnp.float32(0.0)
    lo = 0
    for nf in SPLITS:
        sl = slice(lo, lo + nf)
        lo += nf
        loc_t = loc[sl].transpose(0, 2, 1)
        conf_t = conf[sl].transpose(0, 2, 1)
        lossl, sumpos, lc, npos = _tc_stage(truths_all[sl], anch_t,
                                            loc_t, conf_t)
        sc_part = _sc_stage(lc.reshape(nf, ASC), npos.reshape(nf, 128))
        loss_l = loss_l + lossl[0, 0]
        loss_c = loss_c + sumpos[0, 0] + jnp.sum(sc_part)
    return (loss_l, loss_c)


# speedup vs baseline: 1.0465x; 1.0465x over previous
import functools

import jax
import jax.numpy as jnp
from jax import lax
from jax.experimental import pallas as pl
from jax.experimental.pallas import tpu as pltpu
from jax.experimental.pallas import tpu_sc as plsc

B, F, A, O = 16, 6, 8732, 16
BF = B * F
ASC = 8736
NCHUNK = ASC // 16
NP_RATIO = 3
THRESHOLD = 0.5
VAR0, VAR1 = 0.1, 0.2
NWORKERS = 32
FRAMES_PER_W = BF // NWORKERS


def _tc_body(truths_ref, anch_ref, loc_ref, conf_ref,
             lossl_ref, sumpos_ref, lc_ref, npos_ref, scr_ref):
    f = pl.program_id(0)

    @pl.when(f == 0)
    def _init():
        lossl_ref[...] = jnp.zeros_like(lossl_ref)
        sumpos_ref[...] = jnp.zeros_like(sumpos_ref)
        acx0 = anch_ref[0:1, :]
        acy0 = anch_ref[1:2, :]
        aw0 = anch_ref[2:3, :]
        ah0 = anch_ref[3:4, :]
        scr_ref[0:1, :] = acx0 - aw0 * 0.5
        scr_ref[1:2, :] = acy0 - ah0 * 0.5
        scr_ref[2:3, :] = acx0 + aw0 * 0.5
        scr_ref[3:4, :] = acy0 + ah0 * 0.5
        scr_ref[4:5, :] = aw0 * ah0
        scr_ref[5:6, :] = 1.0 / (VAR0 * aw0)
        scr_ref[6:7, :] = 1.0 / (VAR0 * ah0)
        scr_ref[7:8, :] = 1.0 / aw0
        scr_ref[8:9, :] = 1.0 / ah0

    th = truths_ref[...].reshape(O, 4)
    tx1 = th[:, 0:1]
    ty1 = th[:, 1:2]
    tx2 = th[:, 2:3]
    ty2 = th[:, 3:4]

    acx = anch_ref[0:1, :]
    acy = anch_ref[1:2, :]
    px1 = scr_ref[0:1, :]
    py1 = scr_ref[1:2, :]
    px2 = scr_ref[2:3, :]
    py2 = scr_ref[3:4, :]
    parea = scr_ref[4:5, :]
    inv_vw = scr_ref[5:6, :]
    inv_vh = scr_ref[6:7, :]
    inv_aw = scr_ref[7:8, :]
    inv_ah = scr_ref[8:9, :]

    iw = jnp.maximum(jnp.minimum(tx2, px2) - jnp.maximum(tx1, px1), 0.0)
    ih = jnp.maximum(jnp.minimum(ty2, py2) - jnp.maximum(ty1, py1), 0.0)
    inter = iw * ih
    tarea = (tx2 - tx1) * (ty2 - ty1)
    ov = inter / (tarea + parea - inter)

    bto = jnp.max(ov, axis=0, keepdims=True)
    bti = jnp.argmax(ov, axis=0).astype(jnp.int32)
    bti = bti.reshape(1, A)
    bpi = jnp.argmax(ov, axis=1).astype(jnp.int32)
    bpi = bpi.reshape(O, 1)

    lane16 = lax.broadcasted_iota(jnp.int32, (O, A), 1)
    trange = lax.broadcasted_iota(jnp.int32, (O, 1), 0)
    claimed = lane16 == bpi
    t_last = jnp.max(jnp.where(claimed, trange, -1), axis=0, keepdims=True)
    forced = t_last >= 0
    bto = jnp.where(forced, 2.0, bto)
    bti = jnp.where(forced, t_last, bti)

    pos = bto >= THRESHOLD
    posf = pos.astype(jnp.float32)

    oh = (bti == trange).astype(jnp.float32)
    mc = lax.dot_general(th, oh, (((0,), (0,)), ((), ())),
                         preferred_element_type=jnp.float32)
    mx1 = mc[0:1, :]
    my1 = mc[1:2, :]
    mx2 = mc[2:3, :]
    my2 = mc[3:4, :]

    gcx = ((mx1 + mx2) * 0.5 - acx) * inv_vw
    gcy = ((my1 + my2) * 0.5 - acy) * inv_vh
    gw = jnp.log((mx2 - mx1) * inv_aw) * (1.0 / VAR1)
    gh = jnp.log((my2 - my1) * inv_ah) * (1.0 / VAR1)

    def sl1(x):
        ax = jnp.abs(x)
        return jnp.where(ax < 1.0, 0.5 * x * x, ax - 0.5)

    lc4 = loc_ref[...].reshape(4, A)
    sl = (sl1(lc4[0:1, :] - gcx) + sl1(lc4[1:2, :] - gcy)
          + sl1(lc4[2:3, :] - gw) + sl1(lc4[3:4, :] - gh))
    lossl_ref[...] += jnp.sum(sl * posf, axis=1, keepdims=True)

    cf2 = conf_ref[...].reshape(2, A)
    c0 = cf2[0:1, :]
    c1 = cf2[1:2, :]
    mx = jnp.maximum(c0, c1)
    lse = mx + jnp.log(1.0 + jnp.exp(-jnp.abs(c0 - c1)))
    gathered = jnp.where(pos, c1, c0)
    ce = lse - gathered
    sumpos_ref[...] += jnp.sum(ce * posf, axis=1, keepdims=True)
    lc = jnp.where(pos, 0.0, ce)
    lc_ref[...] = jnp.concatenate(
        [lc, jnp.zeros((1, ASC - A), jnp.float32)], axis=1).reshape(1, 1, ASC)
    npos_ref[...] = jnp.broadcast_to(
        jnp.sum(posf, axis=1, keepdims=True), (1, 128)).reshape(1, 1, 128)


def _tc_stage(truths, anch_t, loc, conf):
    return pl.pallas_call(
        _tc_body,
        grid=(BF,),
        in_specs=[
            pl.BlockSpec((1, O, 4), lambda f: (f, 0, 0)),
            pl.BlockSpec((4, A), lambda f: (0, 0)),
            pl.BlockSpec((1, 4, A), lambda f: (f, 0, 0)),
            pl.BlockSpec((1, 2, A), lambda f: (f, 0, 0)),
        ],
        out_specs=[
            pl.BlockSpec((1, 1), lambda f: (0, 0)),
            pl.BlockSpec((1, 1), lambda f: (0, 0)),
            pl.BlockSpec((1, 1, ASC), lambda f: (f, 0, 0)),
            pl.BlockSpec((1, 1, 128), lambda f: (f, 0, 0)),
        ],
        out_shape=[
            jax.ShapeDtypeStruct((1, 1), jnp.float32),
            jax.ShapeDtypeStruct((1, 1), jnp.float32),
            jax.ShapeDtypeStruct((BF, 1, ASC), jnp.float32),
            jax.ShapeDtypeStruct((BF, 1, 128), jnp.float32),
        ],
        scratch_shapes=[pltpu.VMEM((9, A), jnp.float32)],
    )(truths, anch_t, loc, conf)


def _sc_body(lc_hbm, np_hbm, out_hbm, vbuf, npbuf, outv):
    wid = lax.axis_index("s") * 2 + lax.axis_index("c")
    partial = jnp.zeros((16,), jnp.float32)
    lane0 = lax.broadcasted_iota(jnp.int32, (16,), 0) == 0
    for j in range(FRAMES_PER_W):
        f = wid * FRAMES_PER_W + j
        pltpu.sync_copy(lc_hbm.at[f], vbuf)
        pltpu.sync_copy(np_hbm.at[f], npbuf)
        npos = npbuf[pl.ds(0, 16)].astype(jnp.int32)
        k = jnp.minimum(npos * NP_RATIO, A - 1)

        def max_step(c, acc):
            return jnp.maximum(acc, vbuf[pl.ds(c * 16, 16)])

        vmax = lax.fori_loop(0, NCHUNK, max_step,
                             jnp.zeros((16,), jnp.float32), unroll=8)
        vmax = jnp.full((16,), jnp.max(vmax))

        def bs_step(_, carry):
            lo, hi = carry
            mid = (lo + hi) * 0.5

            def cnt_step(c, acc):
                m = vbuf[pl.ds(c * 16, 16)] >= mid
                return acc + plsc.all_reduce_population_count(m)

            cnt = lax.fori_loop(0, NCHUNK, cnt_step,
                                jnp.zeros((16,), jnp.int32), unroll=16)
            ok = cnt >= k
            lo = jnp.where(ok, mid, lo)
            hi = jnp.where(ok, hi, mid)
            return lo, hi

        lo0 = jnp.zeros((16,), jnp.float32)
        hi0 = vmax + 1.0
        t_f, _ = lax.fori_loop(0, 20, bs_step, (lo0, hi0))

        def fin_step(c, carry):
            cnt_gt, sum_gt = carry
            v = vbuf[pl.ds(c * 16, 16)]
            m = v > t_f
            cnt_gt = cnt_gt + plsc.all_reduce_population_count(m)
            sum_gt = sum_gt + jnp.where(m, v, 0.0)
            return cnt_gt, sum_gt

        cnt_gt, sum_gt = lax.fori_loop(
            0, NCHUNK, fin_step,
            (jnp.zeros((16,), jnp.int32), jnp.zeros((16,), jnp.float32)),
            unroll=8)
        rem = (k - cnt_gt).astype(jnp.float32) * t_f
        partial = partial + sum_gt + jnp.where(lane0, rem, 0.0)

    outv[...] = partial
    pltpu.sync_copy(outv, out_hbm.at[wid])


def _sc_stage(lc, npos):
    mesh = plsc.VectorSubcoreMesh(core_axis_name="c", subcore_axis_name="s")
    kfn = functools.partial(
        pl.kernel,
        mesh=mesh,
        out_type=jax.ShapeDtypeStruct((NWORKERS, 16), jnp.float32),
        scratch_types=[
            pltpu.VMEM((ASC,), jnp.float32),
            pltpu.VMEM((128,), jnp.float32),
            pltpu.VMEM((16,), jnp.float32),
        ],
        compiler_params=pltpu.CompilerParams(needs_layout_passes=False),
    )(_sc_body)
    return kfn(lc, npos)


def kernel(loc_data, conf_data, anchors, targets):
    anch_t = anchors.T
    loc_t = loc_data.reshape(BF, A, 4).transpose(0, 2, 1)
    conf_t = conf_data.reshape(BF, A, 2).transpose(0, 2, 1)
    truths = targets[..., :4].reshape(BF, O, 4)

    lossl, sumpos, lc, npos = _tc_stage(truths, anch_t, loc_t, conf_t)
    sc_part = _sc_stage(lc.reshape(BF, ASC), npos.reshape(BF, 128))

    loss_l = lossl[0, 0]
    loss_c = sumpos[0, 0] + jnp.sum(sc_part)
    return (loss_l, loss_c)


# speedup vs baseline: 1.0469x; 1.0003x over previous
import functools

import jax
import jax.numpy as jnp
from jax import lax
from jax.experimental import pallas as pl
from jax.experimental.pallas import tpu as pltpu
from jax.experimental.pallas import tpu_sc as plsc

B, F, A, O = 16, 6, 8732, 16
BF = B * F
ASC = 8736
NCHUNK = ASC // 16
NP_RATIO = 3
THRESHOLD = 0.5
VAR0, VAR1 = 0.1, 0.2
NWORKERS = 32
FRAMES_PER_W = BF // NWORKERS


def _tc_body(truths_ref, anch_ref, loc_ref, conf_ref,
             lossl_ref, sumpos_ref, lc_ref, npos_ref, scr_ref):
    f = pl.program_id(0)

    @pl.when(f == 0)
    def _init():
        lossl_ref[...] = jnp.zeros_like(lossl_ref)
        sumpos_ref[...] = jnp.zeros_like(sumpos_ref)
        acx0 = anch_ref[0:1, :]
        acy0 = anch_ref[1:2, :]
        aw0 = anch_ref[2:3, :]
        ah0 = anch_ref[3:4, :]
        scr_ref[0:1, :] = acx0 - aw0 * 0.5
        scr_ref[1:2, :] = acy0 - ah0 * 0.5
        scr_ref[2:3, :] = acx0 + aw0 * 0.5
        scr_ref[3:4, :] = acy0 + ah0 * 0.5
        scr_ref[4:5, :] = aw0 * ah0
        scr_ref[5:6, :] = 1.0 / (VAR0 * aw0)
        scr_ref[6:7, :] = 1.0 / (VAR0 * ah0)
        scr_ref[7:8, :] = 1.0 / aw0
        scr_ref[8:9, :] = 1.0 / ah0

    th = truths_ref[...].reshape(O, 4)
    tx1 = th[:, 0:1]
    ty1 = th[:, 1:2]
    tx2 = th[:, 2:3]
    ty2 = th[:, 3:4]

    acx = anch_ref[0:1, :]
    acy = anch_ref[1:2, :]
    px1 = scr_ref[0:1, :]
    py1 = scr_ref[1:2, :]
    px2 = scr_ref[2:3, :]
    py2 = scr_ref[3:4, :]
    parea = scr_ref[4:5, :]
    inv_vw = scr_ref[5:6, :]
    inv_vh = scr_ref[6:7, :]
    inv_aw = scr_ref[7:8, :]
    inv_ah = scr_ref[8:9, :]

    iw = jnp.maximum(jnp.minimum(tx2, px2) - jnp.maximum(tx1, px1), 0.0)
    ih = jnp.maximum(jnp.minimum(ty2, py2) - jnp.maximum(ty1, py1), 0.0)
    inter = iw * ih
    tarea = (tx2 - tx1) * (ty2 - ty1)
    ov = inter / (tarea + parea - inter)

    bto = jnp.max(ov, axis=0, keepdims=True)
    bti = jnp.argmax(ov, axis=0).astype(jnp.int32)
    bti = bti.reshape(1, A)
    bpi = jnp.argmax(ov, axis=1).astype(jnp.int32)
    bpi = bpi.reshape(O, 1)

    lane1 = lax.broadcasted_iota(jnp.int32, (1, A), 1)
    trange = lax.broadcasted_iota(jnp.int32, (O, 1), 0)
    claimed = lane1 == bpi
    t_last = jnp.max(jnp.where(claimed, trange, -1), axis=0, keepdims=True)
    forced = t_last >= 0
    bto = jnp.where(forced, 2.0, bto)
    bti = jnp.where(forced, t_last, bti)

    pos = bto >= THRESHOLD
    posf = pos.astype(jnp.float32)

    oh = (bti == trange).astype(jnp.float32)
    mc = lax.dot_general(th, oh, (((0,), (0,)), ((), ())),
                         preferred_element_type=jnp.float32)
    mx1 = mc[0:1, :]
    my1 = mc[1:2, :]
    mx2 = mc[2:3, :]
    my2 = mc[3:4, :]

    gcx = ((mx1 + mx2) * 0.5 - acx) * inv_vw
    gcy = ((my1 + my2) * 0.5 - acy) * inv_vh
    gw = jnp.log((mx2 - mx1) * inv_aw) * (1.0 / VAR1)
    gh = jnp.log((my2 - my1) * inv_ah) * (1.0 / VAR1)

    def sl1(x):
        ax = jnp.abs(x)
        return jnp.where(ax < 1.0, 0.5 * x * x, ax - 0.5)

    lc4 = loc_ref[...].reshape(4, A)
    sl = (sl1(lc4[0:1, :] - gcx) + sl1(lc4[1:2, :] - gcy)
          + sl1(lc4[2:3, :] - gw) + sl1(lc4[3:4, :] - gh))
    lossl_ref[...] += jnp.sum(sl * posf, axis=1, keepdims=True)

    cf2 = conf_ref[...].reshape(2, A)
    c0 = cf2[0:1, :]
    c1 = cf2[1:2, :]
    mx = jnp.maximum(c0, c1)
    lse = mx + jnp.log(1.0 + jnp.exp(-jnp.abs(c0 - c1)))
    gathered = jnp.where(pos, c1, c0)
    ce = lse - gathered
    sumpos_ref[...] += jnp.sum(ce * posf, axis=1, keepdims=True)
    lc = jnp.where(pos, 0.0, ce)
    lc_ref[...] = jnp.concatenate(
        [lc, jnp.zeros((1, ASC - A), jnp.float32)], axis=1).reshape(1, 1, ASC)
    npos_ref[...] = jnp.broadcast_to(
        jnp.sum(posf, axis=1, keepdims=True), (1, 128)).reshape(1, 1, 128)


def _tc_stage(truths, anch_t, loc, conf):
    return pl.pallas_call(
        _tc_body,
        grid=(BF,),
        in_specs=[
            pl.BlockSpec((1, O, 4), lambda f: (f, 0, 0)),
            pl.BlockSpec((4, A), lambda f: (0, 0)),
            pl.BlockSpec((1, 4, A), lambda f: (f, 0, 0)),
            pl.BlockSpec((1, 2, A), lambda f: (f, 0, 0)),
        ],
        out_specs=[
            pl.BlockSpec((1, 1), lambda f: (0, 0)),
            pl.BlockSpec((1, 1), lambda f: (0, 0)),
            pl.BlockSpec((1, 1, ASC), lambda f: (f, 0, 0)),
            pl.BlockSpec((1, 1, 128), lambda f: (f, 0, 0)),
        ],
        out_shape=[
            jax.ShapeDtypeStruct((1, 1), jnp.float32),
            jax.ShapeDtypeStruct((1, 1), jnp.float32),
            jax.ShapeDtypeStruct((BF, 1, ASC), jnp.float32),
            jax.ShapeDtypeStruct((BF, 1, 128), jnp.float32),
        ],
        scratch_shapes=[pltpu.VMEM((9, A), jnp.float32)],
    )(truths, anch_t, loc, conf)


def _sc_body(lc_hbm, np_hbm, out_hbm, vbuf, npbuf, outv):
    wid = lax.axis_index("s") * 2 + lax.axis_index("c")
    partial = jnp.zeros((16,), jnp.float32)
    lane0 = lax.broadcasted_iota(jnp.int32, (16,), 0) == 0
    for j in range(FRAMES_PER_W):
        f = wid * FRAMES_PER_W + j
        pltpu.sync_copy(lc_hbm.at[f], vbuf)
        pltpu.sync_copy(np_hbm.at[f], npbuf)
        npos = npbuf[pl.ds(0, 16)].astype(jnp.int32)
        k = jnp.minimum(npos * NP_RATIO, A - 1)

        def max_step(c, acc):
            return jnp.maximum(acc, vbuf[pl.ds(c * 16, 16)])

        vmax = lax.fori_loop(0, NCHUNK, max_step,
                             jnp.zeros((16,), jnp.float32), unroll=8)
        vmax = jnp.full((16,), jnp.max(vmax))

        def bs_step(_, carry):
            lo, hi = carry
            mid = (lo + hi) * 0.5

            def cnt_step(c, acc):
                m = vbuf[pl.ds(c * 16, 16)] >= mid
                return acc + plsc.all_reduce_population_count(m)

            cnt = lax.fori_loop(0, NCHUNK, cnt_step,
                                jnp.zeros((16,), jnp.int32), unroll=16)
            ok = cnt >= k
            lo = jnp.where(ok, mid, lo)
            hi = jnp.where(ok, hi, mid)
            return lo, hi

        lo0 = jnp.zeros((16,), jnp.float32)
        hi0 = vmax + 1.0
        t_f, _ = lax.fori_loop(0, 20, bs_step, (lo0, hi0))

        def fin_step(c, carry):
            cnt_gt, sum_gt = carry
            v = vbuf[pl.ds(c * 16, 16)]
            m = v > t_f
            cnt_gt = cnt_gt + plsc.all_reduce_population_count(m)
            sum_gt = sum_gt + jnp.where(m, v, 0.0)
            return cnt_gt, sum_gt

        cnt_gt, sum_gt = lax.fori_loop(
            0, NCHUNK, fin_step,
            (jnp.zeros((16,), jnp.int32), jnp.zeros((16,), jnp.float32)),
            unroll=8)
        rem = (k - cnt_gt).astype(jnp.float32) * t_f
        partial = partial + sum_gt + jnp.where(lane0, rem, 0.0)

    outv[...] = partial
    pltpu.sync_copy(outv, out_hbm.at[wid])


def _sc_stage(lc, npos):
    mesh = plsc.VectorSubcoreMesh(core_axis_name="c", subcore_axis_name="s")
    kfn = functools.partial(
        pl.kernel,
        mesh=mesh,
        out_type=jax.ShapeDtypeStruct((NWORKERS, 16), jnp.float32),
        scratch_types=[
            pltpu.VMEM((ASC,), jnp.float32),
            pltpu.VMEM((128,), jnp.float32),
            pltpu.VMEM((16,), jnp.float32),
        ],
        compiler_params=pltpu.CompilerParams(needs_layout_passes=False),
    )(_sc_body)
    return kfn(lc, npos)


def kernel(loc_data, conf_data, anchors, targets):
    anch_t = anchors.T
    loc_t = loc_data.reshape(BF, A, 4).transpose(0, 2, 1)
    conf_t = conf_data.reshape(BF, A, 2).transpose(0, 2, 1)
    truths = targets[..., :4].reshape(BF, O, 4)

    lossl, sumpos, lc, npos = _tc_stage(truths, anch_t, loc_t, conf_t)
    sc_part = _sc_stage(lc.reshape(BF, ASC), npos.reshape(BF, 128))

    loss_l = lossl[0, 0]
    loss_c = sumpos[0, 0] + jnp.sum(sc_part)
    return (loss_l, loss_c)


# speedup vs baseline: 1.0672x; 1.0193x over previous
import functools

import jax
import jax.numpy as jnp
from jax import lax
from jax.experimental import pallas as pl
from jax.experimental.pallas import tpu as pltpu
from jax.experimental.pallas import tpu_sc as plsc

B, F, A, O = 16, 6, 8732, 16
BF = B * F
ASC = 8736
NCHUNK = ASC // 16
NP_RATIO = 3
THRESHOLD = 0.5
VAR0, VAR1 = 0.1, 0.2
NWORKERS = 32
FRAMES_PER_W = BF // NWORKERS


def _tc_body(truths_ref, anch_ref, loc_ref, conf_ref,
             lossl_ref, sumpos_ref, lc_ref, npos_ref, scr_ref):
    f = pl.program_id(0)

    @pl.when(f == 0)
    def _init():
        lossl_ref[...] = jnp.zeros_like(lossl_ref)
        sumpos_ref[...] = jnp.zeros_like(sumpos_ref)
        acx0 = anch_ref[0:1, :]
        acy0 = anch_ref[1:2, :]
        aw0 = anch_ref[2:3, :]
        ah0 = anch_ref[3:4, :]
        scr_ref[0:1, :] = acx0 - aw0 * 0.5
        scr_ref[1:2, :] = acy0 - ah0 * 0.5
        scr_ref[2:3, :] = acx0 + aw0 * 0.5
        scr_ref[3:4, :] = acy0 + ah0 * 0.5
        scr_ref[4:5, :] = aw0 * ah0
        scr_ref[5:6, :] = 1.0 / (VAR0 * aw0)
        scr_ref[6:7, :] = 1.0 / (VAR0 * ah0)
        scr_ref[7:8, :] = 1.0 / aw0
        scr_ref[8:9, :] = 1.0 / ah0

    th = truths_ref[...].reshape(O, 4)
    tx1 = th[:, 0:1]
    ty1 = th[:, 1:2]
    tx2 = th[:, 2:3]
    ty2 = th[:, 3:4]

    acx = anch_ref[0:1, :]
    acy = anch_ref[1:2, :]
    px1 = scr_ref[0:1, :]
    py1 = scr_ref[1:2, :]
    px2 = scr_ref[2:3, :]
    py2 = scr_ref[3:4, :]
    parea = scr_ref[4:5, :]
    inv_vw = scr_ref[5:6, :]
    inv_vh = scr_ref[6:7, :]
    inv_aw = scr_ref[7:8, :]
    inv_ah = scr_ref[8:9, :]

    iw = jnp.maximum(jnp.minimum(tx2, px2) - jnp.maximum(tx1, px1), 0.0)
    ih = jnp.maximum(jnp.minimum(ty2, py2) - jnp.maximum(ty1, py1), 0.0)
    inter = iw * ih
    tarea = (tx2 - tx1) * (ty2 - ty1)
    ov = inter / (tarea + parea - inter)

    bto = jnp.max(ov, axis=0, keepdims=True)
    bti = jnp.argmax(ov, axis=0).astype(jnp.int32)
    bti = bti.reshape(1, A)

    rowmax = jnp.max(ov, axis=1, keepdims=True)
    trange = lax.broadcasted_iota(jnp.int32, (O, 1), 0)
    claimed = ov == rowmax
    t_last = jnp.max(jnp.where(claimed, trange, -1), axis=0, keepdims=True)
    forced = t_last >= 0
    bto = jnp.where(forced, 2.0, bto)
    bti = jnp.where(forced, t_last, bti)

    pos = bto >= THRESHOLD
    posf = pos.astype(jnp.float32)

    oh = (bti == trange).astype(jnp.float32)
    mc = lax.dot_general(th, oh, (((0,), (0,)), ((), ())),
                         preferred_element_type=jnp.float32)
    mx1 = mc[0:1, :]
    my1 = mc[1:2, :]
    mx2 = mc[2:3, :]
    my2 = mc[3:4, :]

    gcx = ((mx1 + mx2) * 0.5 - acx) * inv_vw
    gcy = ((my1 + my2) * 0.5 - acy) * inv_vh
    gw = jnp.log((mx2 - mx1) * inv_aw) * (1.0 / VAR1)
    gh = jnp.log((my2 - my1) * inv_ah) * (1.0 / VAR1)

    def sl1(x):
        ax = jnp.abs(x)
        return jnp.where(ax < 1.0, 0.5 * x * x, ax - 0.5)

    lc4 = loc_ref[...].reshape(4, A)
    sl = (sl1(lc4[0:1, :] - gcx) + sl1(lc4[1:2, :] - gcy)
          + sl1(lc4[2:3, :] - gw) + sl1(lc4[3:4, :] - gh))
    lossl_ref[...] += jnp.sum(sl * posf, axis=1, keepdims=True)

    cf2 = conf_ref[...].reshape(2, A)
    c0 = cf2[0:1, :]
    c1 = cf2[1:2, :]
    mx = jnp.maximum(c0, c1)
    lse = mx + jnp.log(1.0 + jnp.exp(-jnp.abs(c0 - c1)))
    gathered = jnp.where(pos, c1, c0)
    ce = lse - gathered
    sumpos_ref[...] += jnp.sum(ce * posf, axis=1, keepdims=True)
    lc = jnp.where(pos, 0.0, ce)
    lc_ref[...] = jnp.concatenate(
        [lc, jnp.zeros((1, ASC - A), jnp.float32)], axis=1).reshape(1, 1, ASC)
    npos_ref[...] = jnp.broadcast_to(
        jnp.sum(posf, axis=1, keepdims=True), (1, 128)).reshape(1, 1, 128)


def _tc_stage(truths, anch_t, loc, conf):
    return pl.pallas_call(
        _tc_body,
        grid=(BF,),
        in_specs=[
            pl.BlockSpec((1, O, 4), lambda f: (f, 0, 0)),
            pl.BlockSpec((4, A), lambda f: (0, 0)),
            pl.BlockSpec((1, 4, A), lambda f: (f, 0, 0)),
            pl.BlockSpec((1, 2, A), lambda f: (f, 0, 0)),
        ],
        out_specs=[
            pl.BlockSpec((1, 1), lambda f: (0, 0)),
            pl.BlockSpec((1, 1), lambda f: (0, 0)),
            pl.BlockSpec((1, 1, ASC), lambda f: (f, 0, 0)),
            pl.BlockSpec((1, 1, 128), lambda f: (f, 0, 0)),
        ],
        out_shape=[
            jax.ShapeDtypeStruct((1, 1), jnp.float32),
            jax.ShapeDtypeStruct((1, 1), jnp.float32),
            jax.ShapeDtypeStruct((BF, 1, ASC), jnp.float32),
            jax.ShapeDtypeStruct((BF, 1, 128), jnp.float32),
        ],
        scratch_shapes=[pltpu.VMEM((9, A), jnp.float32)],
    )(truths, anch_t, loc, conf)


def _sc_body(lc_hbm, np_hbm, out_hbm, vbuf, npbuf, outv):
    wid = lax.axis_index("s") * 2 + lax.axis_index("c")
    partial = jnp.zeros((16,), jnp.float32)
    lane0 = lax.broadcasted_iota(jnp.int32, (16,), 0) == 0
    for j in range(FRAMES_PER_W):
        f = wid * FRAMES_PER_W + j
        pltpu.sync_copy(lc_hbm.at[f], vbuf)
        pltpu.sync_copy(np_hbm.at[f], npbuf)
        npos = npbuf[pl.ds(0, 16)].astype(jnp.int32)
        k = jnp.minimum(npos * NP_RATIO, A - 1)

        def max_step(c, acc):
            return jnp.maximum(acc, vbuf[pl.ds(c * 16, 16)])

        vmax = lax.fori_loop(0, NCHUNK, max_step,
                             jnp.zeros((16,), jnp.float32), unroll=8)
        vmax = jnp.full((16,), jnp.max(vmax))

        def bs_step(_, carry):
            lo, hi = carry
            mid = (lo + hi) * 0.5

            def cnt_step(c, acc):
                m = vbuf[pl.ds(c * 16, 16)] >= mid
                return acc + plsc.all_reduce_population_count(m)

            cnt = lax.fori_loop(0, NCHUNK, cnt_step,
                                jnp.zeros((16,), jnp.int32), unroll=16)
            ok = cnt >= k
            lo = jnp.where(ok, mid, lo)
            hi = jnp.where(ok, hi, mid)
            return lo, hi

        lo0 = jnp.zeros((16,), jnp.float32)
        hi0 = vmax + 1.0
        t_f, _ = lax.fori_loop(0, 20, bs_step, (lo0, hi0))

        def fin_step(c, carry):
            cnt_gt, sum_gt = carry
            v = vbuf[pl.ds(c * 16, 16)]
            m = v > t_f
            cnt_gt = cnt_gt + plsc.all_reduce_population_count(m)
            sum_gt = sum_gt + jnp.where(m, v, 0.0)
            return cnt_gt, sum_gt

        cnt_gt, sum_gt = lax.fori_loop(
            0, NCHUNK, fin_step,
            (jnp.zeros((16,), jnp.int32), jnp.zeros((16,), jnp.float32)),
            unroll=8)
        rem = (k - cnt_gt).astype(jnp.float32) * t_f
        partial = partial + sum_gt + jnp.where(lane0, rem, 0.0)

    outv[...] = partial
    pltpu.sync_copy(outv, out_hbm.at[wid])


def _sc_stage(lc, npos):
    mesh = plsc.VectorSubcoreMesh(core_axis_name="c", subcore_axis_name="s")
    kfn = functools.partial(
        pl.kernel,
        mesh=mesh,
        out_type=jax.ShapeDtypeStruct((NWORKERS, 16), jnp.float32),
        scratch_types=[
            pltpu.VMEM((ASC,), jnp.float32),
            pltpu.VMEM((128,), jnp.float32),
            pltpu.VMEM((16,), jnp.float32),
        ],
        compiler_params=pltpu.CompilerParams(needs_layout_passes=False),
    )(_sc_body)
    return kfn(lc, npos)


def kernel(loc_data, conf_data, anchors, targets):
    anch_t = anchors.T
    loc_t = loc_data.reshape(BF, A, 4).transpose(0, 2, 1)
    conf_t = conf_data.reshape(BF, A, 2).transpose(0, 2, 1)
    truths = targets[..., :4].reshape(BF, O, 4)

    lossl, sumpos, lc, npos = _tc_stage(truths, anch_t, loc_t, conf_t)
    sc_part = _sc_stage(lc.reshape(BF, ASC), npos.reshape(BF, 128))

    loss_l = lossl[0, 0]
    loss_c = sumpos[0, 0] + jnp.sum(sc_part)
    return (loss_l, loss_c)
